# baseline (speedup 1.0000x reference)
import functools

import jax
import jax.numpy as jnp
from jax import lax
from jax.experimental import pallas as pl
from jax.experimental.pallas import tpu as pltpu
from jax.experimental.pallas import tpu_sc as plsc

_VOCAB = 32128
_D = 768
_B = 4096 * 200

_C = 64
_NB = 2
_LOOK = 1


@functools.lru_cache(maxsize=None)
def _build_gather():
    info = plsc.get_sparse_core_info()
    nc, ns = info.num_cores, info.num_subcores
    nw = nc * ns
    b_per_w = _B // nw
    n_chunk = b_per_w // _C

    mesh = plsc.VectorSubcoreMesh(core_axis_name="c", subcore_axis_name="s")

    @functools.partial(
        pl.kernel,
        mesh=mesh,
        out_type=jax.ShapeDtypeStruct((_B, _D), jnp.float32),
        scratch_types=[
            pltpu.VMEM((b_per_w,), jnp.int32),
            pltpu.VMEM((_NB, _C, _D), jnp.float32),
        ] + [pltpu.SemaphoreType.DMA] * (2 * _NB),
    )
    def gather_k(table_hbm, idx_hbm, out_hbm, idx_v, rows_v, *sems):
        gs = sems[:_NB]
        ws = sems[_NB:]
        wid = lax.axis_index("s") * nc + lax.axis_index("c")
        base = wid * b_per_w
        pltpu.sync_copy(idx_hbm.at[pl.ds(base, b_per_w)], idx_v)

        def start_gather(g, slot):
            pltpu.make_async_copy(
                table_hbm.at[idx_v.at[pl.ds(g * _C, _C)]],
                rows_v.at[slot], gs[slot]).start()

        def wait_gather(slot):
            pltpu.make_async_copy(
                table_hbm.at[pl.ds(0, _C)], rows_v.at[slot], gs[slot]).wait()

        def start_wb(g, slot):
            pltpu.make_async_copy(
                rows_v.at[slot],
                out_hbm.at[pl.ds(base + g * _C, _C)], ws[slot]).start()

        def wait_wb(slot):
            pltpu.make_async_copy(
                rows_v.at[slot], out_hbm.at[pl.ds(0, _C)], ws[slot]).wait()

        for q in range(_LOOK):
            start_gather(q, q % _NB)

        def step(g, b):
            wait_gather(b)
            start_wb(g, b)
            return g + _LOOK, (b + _LOOK) % _NB

        for b in range(_NB):
            q, slot = step(b, b)
            if q >= _NB:
                wait_wb(slot)
            start_gather(q, slot)

        def body(k, carry):
            for b in range(_NB):
                q, slot = step(k * _NB + b, b)
                wait_wb(slot)
                start_gather(q, slot)
            return carry

        lax.fori_loop(1, n_chunk // _NB - 1, body, 0)

        for b in range(_NB):
            q, slot = step(n_chunk - _NB + b, b)
            if q < n_chunk:
                wait_wb(slot)
                start_gather(q, slot)
        for b in range(_NB):
            wait_wb(b)

    return gather_k


def kernel(x, W):
    idx = x.reshape(-1).astype(jnp.int32)
    out = _build_gather()(W, idx)
    return out.reshape(x.shape + (W.shape[1],))


# speedup vs baseline: 1.0037x; 1.0037x over previous
import functools

import jax
import jax.numpy as jnp
from jax import lax
from jax.experimental import pallas as pl
from jax.experimental.pallas import tpu as pltpu
from jax.experimental.pallas import tpu_sc as plsc

_VOCAB = 32128
_D = 768
_B = 4096 * 200

_C = 16
_NB = 8
_LOOK = 4


@functools.lru_cache(maxsize=None)
def _build_gather():
    info = plsc.get_sparse_core_info()
    nc, ns = info.num_cores, info.num_subcores
    nw = nc * ns
    b_per_w = _B // nw
    n_chunk = b_per_w // _C

    mesh = plsc.VectorSubcoreMesh(core_axis_name="c", subcore_axis_name="s")

    @functools.partial(
        pl.kernel,
        mesh=mesh,
        out_type=jax.ShapeDtypeStruct((_B, _D), jnp.float32),
        scratch_types=[
            pltpu.VMEM((b_per_w,), jnp.int32),
            pltpu.VMEM((_NB, _C, _D), jnp.float32),
        ] + [pltpu.SemaphoreType.DMA] * (2 * _NB),
    )
    def gather_k(table_hbm, idx_hbm, out_hbm, idx_v, rows_v, *sems):
        gs = sems[:_NB]
        ws = sems[_NB:]
        wid = lax.axis_index("s") * nc + lax.axis_index("c")
        base = wid * b_per_w
        pltpu.sync_copy(idx_hbm.at[pl.ds(base, b_per_w)], idx_v)

        def start_gather(g, slot):
            pltpu.make_async_copy(
                table_hbm.at[idx_v.at[pl.ds(g * _C, _C)]],
                rows_v.at[slot], gs[slot]).start()

        def wait_gather(slot):
            pltpu.make_async_copy(
                table_hbm.at[pl.ds(0, _C)], rows_v.at[slot], gs[slot]).wait()

        def start_wb(g, slot):
            pltpu.make_async_copy(
                rows_v.at[slot],
                out_hbm.at[pl.ds(base + g * _C, _C)], ws[slot]).start()

        def wait_wb(slot):
            pltpu.make_async_copy(
                rows_v.at[slot], out_hbm.at[pl.ds(0, _C)], ws[slot]).wait()

        for q in range(_LOOK):
            start_gather(q, q % _NB)

        def step(g, b):
            wait_gather(b)
            start_wb(g, b)
            return g + _LOOK, (b + _LOOK) % _NB

        for b in range(_NB):
            q, slot = step(b, b)
            if q >= _NB:
                wait_wb(slot)
            start_gather(q, slot)

        def body(k, carry):
            for b in range(_NB):
                q, slot = step(k * _NB + b, b)
                wait_wb(slot)
                start_gather(q, slot)
            return carry

        lax.fori_loop(1, n_chunk // _NB - 1, body, 0)

        for b in range(_NB):
            q, slot = step(n_chunk - _NB + b, b)
            if q < n_chunk:
                wait_wb(slot)
                start_gather(q, slot)
        for b in range(_NB):
            wait_wb(b)

    return gather_k


def kernel(x, W):
    idx = x.reshape(-1).astype(jnp.int32)
    out = _build_gather()(W, idx)
    return out.reshape(x.shape + (W.shape[1],))


# speedup vs baseline: 1.0056x; 1.0018x over previous
import functools

import jax
import jax.numpy as jnp
from jax import lax
from jax.experimental import pallas as pl
from jax.experimental.pallas import tpu as pltpu
from jax.experimental.pallas import tpu_sc as plsc

_VOCAB = 32128
_D = 768
_B = 4096 * 200

_C = 32
_NB = 4
_LOOK = 2


@functools.lru_cache(maxsize=None)
def _build_gather():
    info = plsc.get_sparse_core_info()
    nc, ns = info.num_cores, info.num_subcores
    nw = nc * ns
    b_per_w = _B // nw
    n_chunk = b_per_w // _C

    mesh = plsc.VectorSubcoreMesh(core_axis_name="c", subcore_axis_name="s")

    @functools.partial(
        pl.kernel,
        mesh=mesh,
        out_type=jax.ShapeDtypeStruct((_B, _D), jnp.float32),
        scratch_types=[
            pltpu.VMEM((b_per_w,), jnp.int32),
            pltpu.VMEM((_NB, _C, _D), jnp.float32),
        ] + [pltpu.SemaphoreType.DMA] * (2 * _NB),
    )
    def gather_k(table_hbm, idx_hbm, out_hbm, idx_v, rows_v, *sems):
        gs = sems[:_NB]
        ws = sems[_NB:]
        wid = lax.axis_index("s") * nc + lax.axis_index("c")
        base = wid * b_per_w
        pltpu.sync_copy(idx_hbm.at[pl.ds(base, b_per_w)], idx_v)

        def start_gather(g, slot):
            for j in range(_C // 16):
                v = idx_v[pl.ds(g * _C + j * 16, 16)]
                for r in range(16):
                    pltpu.make_async_copy(
                        table_hbm.at[pl.ds(v[r], 1)],
                        rows_v.at[slot, pl.ds(j * 16 + r, 1)],
                        gs[slot]).start()

        def wait_gather(slot):
            pltpu.make_async_copy(
                table_hbm.at[pl.ds(0, _C)], rows_v.at[slot], gs[slot]).wait()

        def start_wb(g, slot):
            pltpu.make_async_copy(
                rows_v.at[slot],
                out_hbm.at[pl.ds(base + g * _C, _C)], ws[slot]).start()

        def wait_wb(slot):
            pltpu.make_async_copy(
                rows_v.at[slot], out_hbm.at[pl.ds(0, _C)], ws[slot]).wait()

        for q in range(_LOOK):
            start_gather(q, q % _NB)

        def step(g, b):
            wait_gather(b)
            start_wb(g, b)
            return g + _LOOK, (b + _LOOK) % _NB

        for b in range(_NB):
            q, slot = step(b, b)
            if q >= _NB:
                wait_wb(slot)
            start_gather(q, slot)

        def body(k, carry):
            for b in range(_NB):
                q, slot = step(k * _NB + b, b)
                wait_wb(slot)
                start_gather(q, slot)
            return carry

        lax.fori_loop(1, n_chunk // _NB - 1, body, 0)

        for b in range(_NB):
            q, slot = step(n_chunk - _NB + b, b)
            if q < n_chunk:
                wait_wb(slot)
                start_gather(q, slot)
        for b in range(_NB):
            wait_wb(b)

    return gather_k


def kernel(x, W):
    idx = x.reshape(-1).astype(jnp.int32)
    out = _build_gather()(W, idx)
    return out.reshape(x.shape + (W.shape[1],))


# speedup vs baseline: 1.0069x; 1.0014x over previous
import functools

import jax
import jax.numpy as jnp
from jax import lax
from jax.experimental import pallas as pl
from jax.experimental.pallas import tpu as pltpu
from jax.experimental.pallas import tpu_sc as plsc

_D = 768
_B = 4096 * 200

_C = 32
_NB = 4
_LOOK = 3


@functools.lru_cache(maxsize=None)
def _build_gather():
    info = plsc.get_sparse_core_info()
    nc, ns = info.num_cores, info.num_subcores
    nw = nc * ns
    b_per_w = _B // nw
    n_chunk = b_per_w // _C

    mesh = plsc.VectorSubcoreMesh(core_axis_name="c", subcore_axis_name="s")

    @functools.partial(
        pl.kernel,
        mesh=mesh,
        out_type=jax.ShapeDtypeStruct((_B, _D), jnp.float32),
        scratch_types=[
            pltpu.VMEM((b_per_w,), jnp.int32),
            pltpu.VMEM((_NB, _C, _D), jnp.float32),
        ] + [pltpu.SemaphoreType.DMA] * (2 * _NB),
    )
    def gather_k(table_hbm, idx_hbm, out_hbm, idx_v, rows_v, *sems):
        gs = sems[:_NB]
        ws = sems[_NB:]
        wid = lax.axis_index("s") * nc + lax.axis_index("c")
        base = wid * b_per_w
        pltpu.sync_copy(idx_hbm.at[pl.ds(base, b_per_w)], idx_v)

        def start_gather(g, slot):
            pltpu.make_async_copy(
                table_hbm.at[idx_v.at[pl.ds(g * _C, _C)]],
                rows_v.at[slot], gs[slot]).start()

        def wait_gather(slot):
            pltpu.make_async_copy(
                table_hbm.at[pl.ds(0, _C)], rows_v.at[slot], gs[slot]).wait()

        def start_wb(g, slot):
            pltpu.make_async_copy(
                rows_v.at[slot],
                out_hbm.at[pl.ds(base + g * _C, _C)], ws[slot]).start()

        def wait_wb(slot):
            pltpu.make_async_copy(
                rows_v.at[slot], out_hbm.at[pl.ds(0, _C)], ws[slot]).wait()

        for q in range(_LOOK):
            start_gather(q, q % _NB)

        def step(g, b):
            wait_gather(b)
            start_wb(g, b)
            return g + _LOOK, (b + _LOOK) % _NB

        for b in range(_NB):
            q, slot = step(b, b)
            if q >= _NB:
                wait_wb(slot)
            start_gather(q, slot)

        def body(k, carry):
            for b in range(_NB):
                q, slot = step(k * _NB + b, b)
                wait_wb(slot)
                start_gather(q, slot)
            return carry

        lax.fori_loop(1, n_chunk // _NB - 1, body, 0)

        for b in range(_NB):
            q, slot = step(n_chunk - _NB + b, b)
            if q < n_chunk:
                wait_wb(slot)
                start_gather(q, slot)
        for b in range(_NB):
            wait_wb(b)

    return gather_k


def kernel(x, W):
    idx = x.reshape(-1).astype(jnp.int32)
    out = _build_gather()(W, idx)
    return out.reshape(x.shape + (W.shape[1],))
